# baseline (speedup 1.0000x reference)
import functools

import jax
import jax.numpy as jnp
from jax import lax
from jax.experimental import pallas as pl
from jax.experimental.pallas import tpu as pltpu
from jax.experimental.pallas import tpu_sc as plsc

_NW = 32
_CH = 12800


def _make_sc(n):
    n_chunks = n // _CH
    mesh = plsc.VectorSubcoreMesh(core_axis_name="c", subcore_axis_name="s",
                                  num_cores=2)

    @functools.partial(
        pl.kernel,
        mesh=mesh,
        out_type=jax.ShapeDtypeStruct((5, n), jnp.int32),
        scratch_types=[
            pltpu.VMEM((16,), jnp.int32),
            pltpu.VMEM((_CH,), jnp.int32),
            pltpu.VMEM((5, _CH), jnp.int32),
        ],
    )
    def sck(par_hbm, sp_hbm, out_hbm, par_v, sp_v, planes_v):
        w = lax.axis_index("s") * 2 + lax.axis_index("c")
        pltpu.sync_copy(par_hbm, par_v)
        nk = (n_chunks - w + _NW - 1) // _NW

        def chunk_body(k, carry):
            off = (w + k * _NW) * _CH
            pltpu.sync_copy(sp_hbm.at[pl.ds(off, _CH)], sp_v)
            par = par_v[...]
            k0 = lax.broadcast_in_dim(par[0:1], (16,), (0,))
            k1 = lax.broadcast_in_dim(par[1:2], (16,), (0,))
            vm = lax.broadcast_in_dim(par[2:3], (16,), (0,))
            one = jnp.ones((16,), jnp.int32)

            def grp(g, carry2):
                sp16 = sp_v[pl.ds(g * 16, 16)]
                hi = sp16 >= 10
                base = jnp.where(hi, sp16 - 10, sp16)
                kv = jnp.where(hi, k1, k0)
                idx = lax.shift_right_logical(kv, base * 3) & 7
                v = lax.shift_left(one, idx) & vm
                for j in range(5):
                    planes_v[j, pl.ds(g * 16, 16)] = (
                        lax.shift_right_logical(v, j) & 1
                    )
                return carry2

            lax.fori_loop(0, _CH // 16, grp, 0)
            pltpu.sync_copy(planes_v, out_hbm.at[:, pl.ds(off, _CH)])
            return carry

        lax.fori_loop(0, nk, chunk_body, 0)

    return sck


def kernel(species, species_to_index, n_species):
    n = species.shape[0]
    s2i = species_to_index.astype(jnp.int32)
    k0 = jnp.sum(s2i[:10] << (jnp.arange(10, dtype=jnp.int32) * 3)).astype(jnp.int32)
    k1 = jnp.sum(s2i[10:] << (jnp.arange(7, dtype=jnp.int32) * 3)).astype(jnp.int32)
    vm = (jnp.int32(1) << jnp.asarray(n_species, jnp.int32)) - 1
    params = jnp.zeros((16,), jnp.int32).at[0].set(k0).at[1].set(k1).at[2].set(vm)
    out_t = _make_sc(n)(params, species)
    return out_t.T


# speedup vs baseline: 2.5214x; 2.5214x over previous
import jax
import jax.numpy as jnp
from jax.experimental import pallas as pl
from jax.experimental.pallas import tpu as pltpu

_C = 80000


def _ohT_kernel(k_ref, sp_ref, out_ref):
    k0 = k_ref[0]
    k1 = k_ref[1]
    vm = k_ref[2]
    sp = sp_ref[0]
    hi = sp >= 10
    base = jnp.where(hi, sp - 10, sp)
    kv = jnp.where(hi, k1, k0)
    idx = jax.lax.shift_right_logical(kv, base * 3) & 7
    v = (jnp.int32(1) << idx) & vm
    j5 = jax.lax.broadcasted_iota(jnp.int32, (5, _C), 0)
    for r in range(8):
        row = jnp.broadcast_to(v[r : r + 1, :], (5, _C))
        out_ref[:, r * _C : (r + 1) * _C] = (
            jax.lax.shift_right_logical(row, j5) & 1
        )


def kernel(species, species_to_index, n_species):
    n = species.shape[0]
    cols = 8 * _C
    nb = n // cols
    sp3 = species.reshape(nb, 8, _C)
    s2i = species_to_index.astype(jnp.int32)
    k0 = jnp.sum(s2i[:10] << (jnp.arange(10, dtype=jnp.int32) * 3)).astype(jnp.int32)
    k1 = jnp.sum(s2i[10:] << (jnp.arange(7, dtype=jnp.int32) * 3)).astype(jnp.int32)
    vm = (jnp.int32(1) << jnp.asarray(n_species, jnp.int32)) - 1
    kparams = jnp.stack([k0, k1, vm])

    out_t = pl.pallas_call(
        _ohT_kernel,
        grid=(nb,),
        in_specs=[
            pl.BlockSpec(memory_space=pltpu.SMEM),
            pl.BlockSpec((1, 8, _C), lambda i: (i, 0, 0)),
        ],
        out_specs=pl.BlockSpec((5, cols), lambda i: (0, i)),
        out_shape=jax.ShapeDtypeStruct((5, n), jnp.int32),
    )(kparams, sp3)
    return out_t.T


# speedup vs baseline: 3.1616x; 1.2539x over previous
import jax
import jax.numpy as jnp
from jax.experimental import pallas as pl
from jax.experimental.pallas import tpu as pltpu

_NB = 10


def _ohT_kernel(k_ref, sp_ref, out_ref):
    k0 = k_ref[0]
    k1 = k_ref[1]
    vm = k_ref[2]
    sp = sp_ref[...]
    r, _ = sp.shape
    hi = sp >= 10
    base = jnp.where(hi, sp - 10, sp)
    kv = jnp.where(hi, k1, k0)
    idx = jax.lax.shift_right_logical(kv, base * 3) & 7
    v = (jnp.int32(1) << idx) & vm
    vrow = v.reshape(1, r * 128)
    j5 = jax.lax.broadcasted_iota(jnp.int32, (5, r * 128), 0)
    b5 = jnp.broadcast_to(vrow, (5, r * 128))
    out_ref[...] = jax.lax.shift_right_logical(b5, j5) & 1


def kernel(species, species_to_index, n_species):
    n = species.shape[0]
    rows = n // 128
    r = rows // _NB
    sp2 = species.reshape(rows, 128)
    s2i = species_to_index.astype(jnp.int32)
    k0 = jnp.sum(s2i[:10] << (jnp.arange(10, dtype=jnp.int32) * 3)).astype(jnp.int32)
    k1 = jnp.sum(s2i[10:] << (jnp.arange(7, dtype=jnp.int32) * 3)).astype(jnp.int32)
    vm = (jnp.int32(1) << jnp.asarray(n_species, jnp.int32)) - 1
    kparams = jnp.stack([k0, k1, vm])

    out_t = pl.pallas_call(
        _ohT_kernel,
        grid=(_NB,),
        in_specs=[
            pl.BlockSpec(memory_space=pltpu.SMEM),
            pl.BlockSpec((r, 128), lambda i: (i, 0)),
        ],
        out_specs=pl.BlockSpec((5, r * 128), lambda i: (0, i)),
        out_shape=jax.ShapeDtypeStruct((5, n), jnp.int32),
    )(kparams, sp2)
    return out_t.T
